# Initial kernel scaffold; baseline (speedup 1.0000x reference)
import jax
import jax.numpy as jnp
from jax.experimental import pallas as pl


def kernel(x, edge_index, etype, ntype, W0, edge_emb0, Wr0, a_l0, a_r0, a_e0, W1, edge_emb1, Wr1, a_l1, a_r1, a_e1, res_W, res_b):
    raise NotImplementedError("write your pallas kernel here")


# speedup vs baseline: 2.9415x; 2.9415x over previous
import jax
import jax.numpy as jnp
from jax.experimental import pallas as pl

_HIGH = jax.lax.Precision.HIGHEST


def _dense0_body(x_ref, w_ref, al_ref, ar_ref, emb_ref, hl_ref, hr_ref):
    emb = jnp.dot(x_ref[...], w_ref[...], precision=_HIGH,
                  preferred_element_type=jnp.float32)
    emb = jnp.where(jnp.isnan(emb), 0.0, emb)
    emb_ref[...] = emb
    hl_ref[...] = jnp.dot(emb, al_ref[...], precision=_HIGH,
                          preferred_element_type=jnp.float32)
    hr_ref[...] = jnp.dot(emb, ar_ref[...], precision=_HIGH,
                          preferred_element_type=jnp.float32)


def _dense1_body(h_ref, w_ref, rw_ref, rb_ref, al_ref, ar_ref,
                 emb_ref, res_ref, hl_ref, hr_ref):
    h = h_ref[...]
    h = jnp.where(h > 0, h, jnp.exp(jnp.minimum(h, 0.0)) - 1.0)
    emb = jnp.dot(h, w_ref[...], precision=_HIGH,
                  preferred_element_type=jnp.float32)
    emb = jnp.where(jnp.isnan(emb), 0.0, emb)
    emb_ref[...] = emb
    res_ref[...] = jnp.dot(h, rw_ref[...], precision=_HIGH,
                           preferred_element_type=jnp.float32) + rb_ref[...]
    hl_ref[...] = jnp.dot(emb, al_ref[...], precision=_HIGH,
                          preferred_element_type=jnp.float32)
    hr_ref[...] = jnp.dot(emb, ar_ref[...], precision=_HIGH,
                          preferred_element_type=jnp.float32)


def _att_body(a_ref, b_ref, c_ref, o_ref):
    s = a_ref[...] + b_ref[...] + c_ref[...]
    o_ref[...] = jnp.where(s >= 0, s, 0.2 * s)


def _exp_body(att_ref, m_ref, o_ref):
    o_ref[...] = jnp.exp(att_ref[...] - m_ref[...])


def _msg_body(embr_ref, e_ref, s_ref, k_ref, o_ref):
    alpha = e_ref[...] / (s_ref[...] + 1e-9)
    aw = jnp.dot(alpha, k_ref[...], precision=_HIGH,
                 preferred_element_type=jnp.float32)
    o_ref[...] = embr_ref[...] * aw


def _dense0(x, w, al, ar, bn):
    n, din = x.shape
    dout = w.shape[1]
    h = al.shape[1]
    return pl.pallas_call(
        _dense0_body,
        grid=(n // bn,),
        in_specs=[
            pl.BlockSpec((bn, din), lambda i: (i, 0)),
            pl.BlockSpec((din, dout), lambda i: (0, 0)),
            pl.BlockSpec((dout, h), lambda i: (0, 0)),
            pl.BlockSpec((dout, h), lambda i: (0, 0)),
        ],
        out_specs=[
            pl.BlockSpec((bn, dout), lambda i: (i, 0)),
            pl.BlockSpec((bn, h), lambda i: (i, 0)),
            pl.BlockSpec((bn, h), lambda i: (i, 0)),
        ],
        out_shape=[
            jax.ShapeDtypeStruct((n, dout), jnp.float32),
            jax.ShapeDtypeStruct((n, h), jnp.float32),
            jax.ShapeDtypeStruct((n, h), jnp.float32),
        ],
    )(x, w, al, ar)


def _dense1(hin, w, rw, rb, al, ar, bn):
    n, din = hin.shape
    dout = w.shape[1]
    h = al.shape[1]
    return pl.pallas_call(
        _dense1_body,
        grid=(n // bn,),
        in_specs=[
            pl.BlockSpec((bn, din), lambda i: (i, 0)),
            pl.BlockSpec((din, dout), lambda i: (0, 0)),
            pl.BlockSpec((din, dout), lambda i: (0, 0)),
            pl.BlockSpec((1, dout), lambda i: (0, 0)),
            pl.BlockSpec((dout, h), lambda i: (0, 0)),
            pl.BlockSpec((dout, h), lambda i: (0, 0)),
        ],
        out_specs=[
            pl.BlockSpec((bn, dout), lambda i: (i, 0)),
            pl.BlockSpec((bn, dout), lambda i: (i, 0)),
            pl.BlockSpec((bn, h), lambda i: (i, 0)),
            pl.BlockSpec((bn, h), lambda i: (i, 0)),
        ],
        out_shape=[
            jax.ShapeDtypeStruct((n, dout), jnp.float32),
            jax.ShapeDtypeStruct((n, dout), jnp.float32),
            jax.ShapeDtypeStruct((n, h), jnp.float32),
            jax.ShapeDtypeStruct((n, h), jnp.float32),
        ],
    )(hin, w, rw, rb, al, ar)


def _att(a, b, c, be):
    e, h = a.shape
    spec = pl.BlockSpec((be, h), lambda i: (i, 0))
    return pl.pallas_call(
        _att_body,
        grid=(e // be,),
        in_specs=[spec, spec, spec],
        out_specs=spec,
        out_shape=jax.ShapeDtypeStruct((e, h), jnp.float32),
    )(a, b, c)


def _expk(att, mcol, be):
    e, h = att.shape
    spec = pl.BlockSpec((be, h), lambda i: (i, 0))
    return pl.pallas_call(
        _exp_body,
        grid=(e // be,),
        in_specs=[spec, spec],
        out_specs=spec,
        out_shape=jax.ShapeDtypeStruct((e, h), jnp.float32),
    )(att, mcol)


def _msg(embr, ek, scol, k, be):
    e, d = embr.shape
    h = ek.shape[1]
    return pl.pallas_call(
        _msg_body,
        grid=(e // be,),
        in_specs=[
            pl.BlockSpec((be, d), lambda i: (i, 0)),
            pl.BlockSpec((be, h), lambda i: (i, 0)),
            pl.BlockSpec((be, h), lambda i: (i, 0)),
            pl.BlockSpec((h, d), lambda i: (0, 0)),
        ],
        out_specs=pl.BlockSpec((be, d), lambda i: (i, 0)),
        out_shape=jax.ShapeDtypeStruct((e, d), jnp.float32),
    )(embr, ek, scol, k)


def _etype_table(edge_emb, wr, a_e):
    t, ed = edge_emb.shape
    h = a_e.shape[1]
    ee = jnp.einsum('td,tdk->tk', edge_emb, wr,
                    precision=_HIGH).reshape(t, h, ed)
    return (a_e[0][None] * ee).sum(-1)


def kernel(x, edge_index, etype, ntype, W0, edge_emb0, Wr0, a_l0, a_r0, a_e0,
           W1, edge_emb1, Wr1, a_l1, a_r1, a_e1, res_W, res_b):
    n = x.shape[0]
    e = edge_index.shape[1]
    h0, d0 = a_l0.shape[1], a_l0.shape[2]
    h1, d1 = a_l1.shape[1], a_l1.shape[2]
    row = edge_index[0]
    col = edge_index[1]

    eye0 = jnp.eye(h0, dtype=jnp.float32)
    al0 = (a_l0[0][:, :, None] * eye0[:, None, :]).reshape(h0 * d0, h0)
    ar0 = (a_r0[0][:, :, None] * eye0[:, None, :]).reshape(h0 * d0, h0)
    k0 = jnp.repeat(eye0, d0, axis=1)
    al1 = a_l1[0].reshape(h1, d1).T
    ar1 = a_r1[0].reshape(h1, d1).T
    k1 = jnp.ones((h1, d1), jnp.float32)
    r = jnp.arange(h0 * d0)
    perm = (r % d0) * h0 + r // d0
    w1p = W1[perm]
    rwp = res_W[perm]
    rb2 = res_b.reshape(1, -1)

    he0_t = _etype_table(edge_emb0, Wr0, a_e0)
    he1_t = _etype_table(edge_emb1, Wr1, a_e1)

    emb0, hl0, hr0 = _dense0(x, W0, al0, ar0, bn=1000)
    att0 = _att(hl0[row], hr0[col], he0_t[etype], be=4000)
    m0 = jax.ops.segment_max(att0, col, num_segments=n)
    m0 = jnp.where(jnp.isfinite(m0), m0, 0.0)
    e0 = _expk(att0, m0[col], be=4000)
    s0 = jax.ops.segment_sum(e0, col, num_segments=n)
    msg0 = _msg(emb0[row], e0, s0[col], k0, be=2000)
    hmid = jax.ops.segment_sum(msg0, col, num_segments=n)

    emb1, res, hl1, hr1 = _dense1(hmid, w1p, rwp, rb2, al1, ar1, bn=1000)
    att1 = _att(hl1[row], hr1[col], he1_t[etype], be=4000)
    m1 = jax.ops.segment_max(att1, col, num_segments=n)
    m1 = jnp.where(jnp.isfinite(m1), m1, 0.0)
    e1 = _expk(att1, m1[col], be=4000)
    s1 = jax.ops.segment_sum(e1, col, num_segments=n)
    msg1 = _msg(emb1[row], e1, s1[col], k1, be=4000)
    agg1 = jax.ops.segment_sum(msg1, col, num_segments=n)

    return agg1 + res


# speedup vs baseline: 2.9487x; 1.0024x over previous
import jax
import jax.numpy as jnp
from jax.experimental import pallas as pl

_HIGH = jax.lax.Precision.HIGHEST


def _dense0_body(x_ref, w_ref, al_ref, ar_ref, emb_ref, hl_ref, hr_ref):
    emb = jnp.dot(x_ref[...], w_ref[...], precision=_HIGH,
                  preferred_element_type=jnp.float32)
    emb = jnp.where(jnp.isnan(emb), 0.0, emb)
    emb_ref[...] = emb
    hl_ref[...] = jnp.dot(emb, al_ref[...], precision=_HIGH,
                          preferred_element_type=jnp.float32)
    hr_ref[...] = jnp.dot(emb, ar_ref[...], precision=_HIGH,
                          preferred_element_type=jnp.float32)


def _dense1_body(h_ref, w_ref, rw_ref, rb_ref, al_ref, ar_ref,
                 emb_ref, res_ref, hl_ref, hr_ref):
    h = h_ref[...]
    h = jnp.where(h > 0, h, jnp.exp(jnp.minimum(h, 0.0)) - 1.0)
    emb = jnp.dot(h, w_ref[...], precision=_HIGH,
                  preferred_element_type=jnp.float32)
    emb = jnp.where(jnp.isnan(emb), 0.0, emb)
    emb_ref[...] = emb
    res_ref[...] = jnp.dot(h, rw_ref[...], precision=_HIGH,
                           preferred_element_type=jnp.float32) + rb_ref[...]
    hl_ref[...] = jnp.dot(emb, al_ref[...], precision=_HIGH,
                          preferred_element_type=jnp.float32)
    hr_ref[...] = jnp.dot(emb, ar_ref[...], precision=_HIGH,
                          preferred_element_type=jnp.float32)


def _att_body(a_ref, b_ref, c_ref, o_ref):
    s = a_ref[...] + b_ref[...] + c_ref[...]
    o_ref[...] = jnp.where(s >= 0, s, 0.2 * s)


def _exp_body(att_ref, m_ref, o_ref):
    o_ref[...] = jnp.exp(att_ref[...] - m_ref[...])


def _msg_body(embr_ref, e_ref, s_ref, k_ref, o_ref):
    alpha = e_ref[...] / (s_ref[...] + 1e-9)
    aw = jnp.dot(alpha, k_ref[...], precision=_HIGH,
                 preferred_element_type=jnp.float32)
    o_ref[...] = embr_ref[...] * aw


def _dense0(x, w, al, ar, bn):
    n, din = x.shape
    dout = w.shape[1]
    h = al.shape[1]
    return pl.pallas_call(
        _dense0_body,
        grid=(n // bn,),
        in_specs=[
            pl.BlockSpec((bn, din), lambda i: (i, 0)),
            pl.BlockSpec((din, dout), lambda i: (0, 0)),
            pl.BlockSpec((dout, h), lambda i: (0, 0)),
            pl.BlockSpec((dout, h), lambda i: (0, 0)),
        ],
        out_specs=[
            pl.BlockSpec((bn, dout), lambda i: (i, 0)),
            pl.BlockSpec((bn, h), lambda i: (i, 0)),
            pl.BlockSpec((bn, h), lambda i: (i, 0)),
        ],
        out_shape=[
            jax.ShapeDtypeStruct((n, dout), jnp.float32),
            jax.ShapeDtypeStruct((n, h), jnp.float32),
            jax.ShapeDtypeStruct((n, h), jnp.float32),
        ],
    )(x, w, al, ar)


def _dense1(hin, w, rw, rb, al, ar, bn):
    n, din = hin.shape
    dout = w.shape[1]
    h = al.shape[1]
    return pl.pallas_call(
        _dense1_body,
        grid=(n // bn,),
        in_specs=[
            pl.BlockSpec((bn, din), lambda i: (i, 0)),
            pl.BlockSpec((din, dout), lambda i: (0, 0)),
            pl.BlockSpec((din, dout), lambda i: (0, 0)),
            pl.BlockSpec((1, dout), lambda i: (0, 0)),
            pl.BlockSpec((dout, h), lambda i: (0, 0)),
            pl.BlockSpec((dout, h), lambda i: (0, 0)),
        ],
        out_specs=[
            pl.BlockSpec((bn, dout), lambda i: (i, 0)),
            pl.BlockSpec((bn, dout), lambda i: (i, 0)),
            pl.BlockSpec((bn, h), lambda i: (i, 0)),
            pl.BlockSpec((bn, h), lambda i: (i, 0)),
        ],
        out_shape=[
            jax.ShapeDtypeStruct((n, dout), jnp.float32),
            jax.ShapeDtypeStruct((n, dout), jnp.float32),
            jax.ShapeDtypeStruct((n, h), jnp.float32),
            jax.ShapeDtypeStruct((n, h), jnp.float32),
        ],
    )(hin, w, rw, rb, al, ar)


def _att(a, b, c, be):
    e, h = a.shape
    spec = pl.BlockSpec((be, h), lambda i: (i, 0))
    return pl.pallas_call(
        _att_body,
        grid=(e // be,),
        in_specs=[spec, spec, spec],
        out_specs=spec,
        out_shape=jax.ShapeDtypeStruct((e, h), jnp.float32),
    )(a, b, c)


def _expk(att, mcol, be):
    e, h = att.shape
    spec = pl.BlockSpec((be, h), lambda i: (i, 0))
    return pl.pallas_call(
        _exp_body,
        grid=(e // be,),
        in_specs=[spec, spec],
        out_specs=spec,
        out_shape=jax.ShapeDtypeStruct((e, h), jnp.float32),
    )(att, mcol)


def _msg(embr, ek, scol, k, be):
    e, d = embr.shape
    h = ek.shape[1]
    return pl.pallas_call(
        _msg_body,
        grid=(e // be,),
        in_specs=[
            pl.BlockSpec((be, d), lambda i: (i, 0)),
            pl.BlockSpec((be, h), lambda i: (i, 0)),
            pl.BlockSpec((be, h), lambda i: (i, 0)),
            pl.BlockSpec((h, d), lambda i: (0, 0)),
        ],
        out_specs=pl.BlockSpec((be, d), lambda i: (i, 0)),
        out_shape=jax.ShapeDtypeStruct((e, d), jnp.float32),
    )(embr, ek, scol, k)


def _etype_table(edge_emb, wr, a_e):
    t, ed = edge_emb.shape
    h = a_e.shape[1]
    ee = jnp.einsum('td,tdk->tk', edge_emb, wr,
                    precision=_HIGH).reshape(t, h, ed)
    return (a_e[0][None] * ee).sum(-1)


def kernel(x, edge_index, etype, ntype, W0, edge_emb0, Wr0, a_l0, a_r0, a_e0,
           W1, edge_emb1, Wr1, a_l1, a_r1, a_e1, res_W, res_b):
    n = x.shape[0]
    e = edge_index.shape[1]
    h0, d0 = a_l0.shape[1], a_l0.shape[2]
    h1, d1 = a_l1.shape[1], a_l1.shape[2]
    row = edge_index[0]
    col = edge_index[1]

    eye0 = jnp.eye(h0, dtype=jnp.float32)
    al0 = (a_l0[0][:, :, None] * eye0[:, None, :]).reshape(h0 * d0, h0)
    ar0 = (a_r0[0][:, :, None] * eye0[:, None, :]).reshape(h0 * d0, h0)
    k0 = jnp.repeat(eye0, d0, axis=1)
    al1 = a_l1[0].reshape(h1, d1).T
    ar1 = a_r1[0].reshape(h1, d1).T
    k1 = jnp.ones((h1, d1), jnp.float32)
    r = jnp.arange(h0 * d0)
    perm = (r % d0) * h0 + r // d0
    w1p = W1[perm]
    rwp = res_W[perm]
    rb2 = res_b.reshape(1, -1)

    he0_t = _etype_table(edge_emb0, Wr0, a_e0)
    he1_t = _etype_table(edge_emb1, Wr1, a_e1)

    emb0, hl0, hr0 = _dense0(x, W0, al0, ar0, bn=2000)
    att0 = _att(hl0[row], hr0[col], he0_t[etype], be=4000)
    m0 = jax.ops.segment_max(att0, col, num_segments=n)
    m0 = jnp.where(jnp.isfinite(m0), m0, 0.0)
    e0 = _expk(att0, m0[col], be=4000)
    s0 = jax.ops.segment_sum(e0, col, num_segments=n)
    msg0 = _msg(emb0[row], e0, s0[col], k0, be=4000)
    hmid = jax.ops.segment_sum(msg0, col, num_segments=n)

    emb1, res, hl1, hr1 = _dense1(hmid, w1p, rwp, rb2, al1, ar1, bn=1000)
    att1 = _att(hl1[row], hr1[col], he1_t[etype], be=4000)
    m1 = jax.ops.segment_max(att1, col, num_segments=n)
    m1 = jnp.where(jnp.isfinite(m1), m1, 0.0)
    e1 = _expk(att1, m1[col], be=4000)
    s1 = jax.ops.segment_sum(e1, col, num_segments=n)
    msg1 = _msg(emb1[row], e1, s1[col], k1, be=4000)
    agg1 = jax.ops.segment_sum(msg1, col, num_segments=n)

    return agg1 + res
